# baseline (speedup 1.0000x reference)
import jax
import jax.numpy as jnp
from jax import lax
from jax.experimental import pallas as pl
from jax.experimental.pallas import tpu as pltpu
from jax.experimental.pallas import tpu_sc as plsc

_NC = 2
_NS = 16
_NW = _NC * _NS

_CHUNK = 128
_NBUF = 4


def _make_sc_gather(num_rows: int, dim: int):
    assert num_rows % (_NW * _CHUNK) == 0
    rows_per_w = num_rows // _NW
    nsteps = rows_per_w // _CHUNK
    assert nsteps % _NBUF == 0
    outer = nsteps // _NBUF

    mesh = plsc.VectorSubcoreMesh(core_axis_name="c", subcore_axis_name="s")

    scratch = [
        pltpu.VMEM((nsteps, _CHUNK), jnp.int32),
        pltpu.VMEM((_NBUF, _CHUNK, dim), jnp.float32),
    ] + [pltpu.SemaphoreType.DMA] * (2 * _NBUF)

    def body(idx_hbm, table_hbm, out_hbm, idx_v, rows_v, *sems):
        gsem = sems[:_NBUF]
        ssem = sems[_NBUF:]
        wid = lax.axis_index("s") * _NC + lax.axis_index("c")
        row_base = wid * rows_per_w

        pltpu.sync_copy(idx_hbm.at[pl.ds(wid * nsteps, nsteps)], idx_v)

        def gather_start(j, b):
            pltpu.async_copy(table_hbm.at[idx_v.at[j]], rows_v.at[b], gsem[b])

        def gather_wait(j, b):
            pltpu.make_async_copy(
                table_hbm.at[idx_v.at[j]], rows_v.at[b], gsem[b]).wait()

        def scatter_start(j, b):
            pltpu.async_copy(
                rows_v.at[b],
                out_hbm.at[pl.ds(row_base + j * _CHUNK, _CHUNK)],
                ssem[b])

        def scatter_wait(j, b):
            pltpu.make_async_copy(
                rows_v.at[b],
                out_hbm.at[pl.ds(row_base + j * _CHUNK, _CHUNK)],
                ssem[b]).wait()

        for b in range(_NBUF):
            gather_start(b, b)
            if b >= 1:
                gather_wait(b - 1, b - 1)
                scatter_start(b - 1, b - 1)

        def outer_body(g, _):
            for b in range(_NBUF):
                j = g * _NBUF + b
                scatter_wait(j - _NBUF, b)
                gather_start(j, b)
                pb = (b - 1) % _NBUF
                gather_wait(j - 1, pb)
                scatter_start(j - 1, pb)
            return 0

        lax.fori_loop(1, outer, outer_body, 0)

        last = nsteps - 1
        bl = last % _NBUF
        gather_wait(last, bl)
        scatter_start(last, bl)
        for b in range(_NBUF):
            j = nsteps - _NBUF + b
            scatter_wait(j, b)

    return pl.kernel(
        body,
        out_type=jax.ShapeDtypeStruct((num_rows, dim), jnp.float32),
        mesh=mesh,
        scratch_types=scratch,
        compiler_params=pltpu.CompilerParams(use_tc_tiling_on_sc=False),
    )


@jax.jit
def kernel(token_ids, weight):
    bsz, seq = token_ids.shape
    num, dim = weight.shape
    idx = token_ids.reshape(-1).astype(jnp.int32)
    num_rows = idx.shape[0]
    nsteps = num_rows // (_NW * _CHUNK)
    idx2d = idx.reshape(_NW * nsteps, _CHUNK)
    out = _make_sc_gather(num_rows, dim)(idx2d, weight)
    return out.reshape(bsz, seq, dim)


# speedup vs baseline: 1.0088x; 1.0088x over previous
import jax
import jax.numpy as jnp
from jax import lax
from jax.experimental import pallas as pl
from jax.experimental.pallas import tpu as pltpu
from jax.experimental.pallas import tpu_sc as plsc

_NC = 2
_NS = 16
_NW = _NC * _NS

_CHUNK = 128
_NBUF = 8
_LAG = 6


def _make_sc_gather(num_rows: int, dim: int):
    assert num_rows % (_NW * _CHUNK) == 0
    rows_per_w = num_rows // _NW
    nsteps = rows_per_w // _CHUNK
    assert nsteps % _NBUF == 0
    outer = nsteps // _NBUF

    mesh = plsc.VectorSubcoreMesh(core_axis_name="c", subcore_axis_name="s")

    scratch = [
        pltpu.VMEM((nsteps, _CHUNK), jnp.int32),
        pltpu.VMEM((_NBUF, _CHUNK, dim), jnp.float32),
    ] + [pltpu.SemaphoreType.DMA] * (2 * _NBUF)

    def body(idx_hbm, table_hbm, out_hbm, idx_v, rows_v, *sems):
        gsem = sems[:_NBUF]
        ssem = sems[_NBUF:]
        wid = lax.axis_index("s") * _NC + lax.axis_index("c")
        row_base = wid * rows_per_w

        pltpu.sync_copy(idx_hbm.at[pl.ds(wid * nsteps, nsteps)], idx_v)

        def gather_start(j, b):
            pltpu.async_copy(table_hbm.at[idx_v.at[j]], rows_v.at[b], gsem[b])

        def gather_wait(j, b):
            pltpu.make_async_copy(
                table_hbm.at[idx_v.at[j]], rows_v.at[b], gsem[b]).wait()

        def scatter_start(j, b):
            pltpu.async_copy(
                rows_v.at[b],
                out_hbm.at[pl.ds(row_base + j * _CHUNK, _CHUNK)],
                ssem[b])

        def scatter_wait(j, b):
            pltpu.make_async_copy(
                rows_v.at[b],
                out_hbm.at[pl.ds(row_base + j * _CHUNK, _CHUNK)],
                ssem[b]).wait()

        for j in range(_NBUF):
            gather_start(j, j)
            if j >= _LAG:
                jc = j - _LAG
                gather_wait(jc, jc)
                scatter_start(jc, jc)

        def outer_body(g, _):
            for b in range(_NBUF):
                j = g * _NBUF + b
                scatter_wait(j - _NBUF, b)
                gather_start(j, b)
                jc = j - _LAG
                bc = (b - _LAG) % _NBUF
                gather_wait(jc, bc)
                scatter_start(jc, bc)
            return 0

        lax.fori_loop(1, outer, outer_body, 0)

        for jc in range(nsteps - _LAG, nsteps):
            gather_wait(jc, jc % _NBUF)
            scatter_start(jc, jc % _NBUF)
        for j in range(nsteps - _NBUF, nsteps):
            scatter_wait(j, j % _NBUF)

    return pl.kernel(
        body,
        out_type=jax.ShapeDtypeStruct((num_rows, dim), jnp.float32),
        mesh=mesh,
        scratch_types=scratch,
        compiler_params=pltpu.CompilerParams(use_tc_tiling_on_sc=False),
    )


@jax.jit
def kernel(token_ids, weight):
    bsz, seq = token_ids.shape
    num, dim = weight.shape
    idx = token_ids.reshape(-1).astype(jnp.int32)
    num_rows = idx.shape[0]
    nsteps = num_rows // (_NW * _CHUNK)
    idx2d = idx.reshape(_NW * nsteps, _CHUNK)
    out = _make_sc_gather(num_rows, dim)(idx2d, weight)
    return out.reshape(bsz, seq, dim)
